# baseline (speedup 1.0000x reference)
import functools

import jax
import jax.numpy as jnp
from jax import lax
from jax.experimental import pallas as pl
from jax.experimental.pallas import tpu as pltpu
from jax.experimental.pallas import tpu_sc as plsc

N = 10000
E = 320000
D = 128
H1 = 32
H2 = 16
HL = 8

NC = 2
NS = 16
NW = NC * NS
NP = 10240
RPT = NP // NS
K = 128
CH = 80
EPT = CH * K
EP = NW * EPT
G = 8
NG = CH // G
SR = N // NS

_SC_PARAMS = pltpu.CompilerParams(use_tc_tiling_on_sc=False)


def _sc_mesh():
    return plsc.VectorSubcoreMesh(
        core_axis_name="c", subcore_axis_name="s", num_cores=NC, num_subcores=NS
    )


def _zero_flat(ref, nwords):
    z = jnp.zeros((16,), jnp.float32)

    def body(i, carry):
        ref[pl.ds(i * 16, 16)] = z
        return carry

    lax.fori_loop(0, nwords // 16, body, 0)


def _zero_2d(ref, rows, cols):
    z = jnp.zeros((16,), jnp.float32)

    def body(r, carry):
        for k in range(cols // 16):
            ref[r, pl.ds(k * 16, 16)] = z
        return carry

    lax.fori_loop(0, rows, body, 0)


@functools.cache
def _build_sc_degree():
    return functools.partial(
        pl.kernel,
        out_type=(
            jax.ShapeDtypeStruct((NP,), jnp.float32),
            jax.ShapeDtypeStruct((NP,), jnp.float32),
        ),
        mesh=_sc_mesh(),
        scratch_types=[
            pltpu.VMEM((CH, K), jnp.int32),
            pltpu.VMEM((K,), jnp.float32),
            pltpu.VMEM((RPT,), jnp.float32),
            pltpu.VMEM_SHARED((NP,), jnp.float32),
        ],
        compiler_params=_SC_PARAMS,
    )(_sc_degree_body)


def _sc_degree_body(ei_hbm, out0, out1, col_v, ones_v, stage_v, acc_sh):
    c = lax.axis_index("c")
    s = lax.axis_index("s")
    w = c * NS + s
    pltpu.sync_copy(ei_hbm.at[1, w], col_v)
    one = jnp.ones((16,), jnp.float32)
    for k in range(K // 16):
        ones_v[pl.ds(k * 16, 16)] = one
    _zero_flat(stage_v, RPT)
    pltpu.sync_copy(stage_v, acc_sh.at[pl.ds(s * RPT, RPT)])
    plsc.subcore_barrier()

    def body(j, carry):
        pltpu.sync_copy(ones_v, acc_sh.at[col_v.at[j]], add=True)
        return carry

    lax.fori_loop(0, CH, body, 0)
    plsc.subcore_barrier()
    pltpu.sync_copy(acc_sh.at[pl.ds(s * RPT, RPT)], stage_v)

    @pl.when(c == 0)
    def _():
        pltpu.sync_copy(stage_v, out0.at[pl.ds(s * RPT, RPT)])

    @pl.when(c == 1)
    def _():
        pltpu.sync_copy(stage_v, out1.at[pl.ds(s * RPT, RPT)])


def _sc_degree(ei):
    return _build_sc_degree()(ei)


@functools.cache
def _make_msgpass(F, stage_tbl):
    scratch = [
        pltpu.VMEM((CH, K), jnp.int32),
        pltpu.VMEM((CH, K), jnp.int32),
        pltpu.VMEM((2, G, K, F), jnp.float32),
        pltpu.VMEM((RPT, F), jnp.float32),
        pltpu.VMEM_SHARED((NP, F), jnp.float32),
        [[pltpu.SemaphoreType.DMA] * G, [pltpu.SemaphoreType.DMA] * G],
    ]
    if stage_tbl:
        scratch.append(pltpu.VMEM_SHARED((N, F), jnp.float32))

    @functools.partial(
        pl.kernel,
        out_type=(
            jax.ShapeDtypeStruct((NP, F), jnp.float32),
            jax.ShapeDtypeStruct((NP, F), jnp.float32),
        ),
        mesh=_sc_mesh(),
        scratch_types=scratch,
        compiler_params=_SC_PARAMS,
        name=f"sc_msgpass_f{F}",
    )
    def _mp(h_hbm, ei_hbm, out0, out1, row_v, col_v, gbuf, stage_v, acc_sh,
            gsems, *maybe_tbl):
        c = lax.axis_index("c")
        s = lax.axis_index("s")
        w = c * NS + s
        pltpu.sync_copy(ei_hbm.at[0, w], row_v)
        pltpu.sync_copy(ei_hbm.at[1, w], col_v)
        if stage_tbl:
            tbl = maybe_tbl[0]
            pltpu.sync_copy(
                h_hbm.at[pl.ds(s * SR, SR)], stage_v.at[pl.ds(0, SR)]
            )
            pltpu.sync_copy(
                stage_v.at[pl.ds(0, SR)], tbl.at[pl.ds(s * SR, SR)]
            )
        else:
            tbl = h_hbm
        _zero_2d(stage_v, RPT, F)
        pltpu.sync_copy(stage_v, acc_sh.at[pl.ds(s * RPT, RPT)])
        plsc.subcore_barrier()
        for b in range(G):
            pltpu.async_copy(tbl.at[row_v.at[b]], gbuf.at[0, b], gsems[0][b])

        def body(i, carry):
            for p in (0, 1):
                g = i * 2 + p
                q = 1 - p

                @pl.when(g + 1 < NG)
                def _():
                    for b in range(G):
                        pltpu.async_copy(
                            tbl.at[row_v.at[(g + 1) * G + b]],
                            gbuf.at[q, b], gsems[q][b],
                        )

                for b in range(G):
                    pltpu.make_async_copy(
                        tbl.at[row_v.at[g * G + b]], gbuf.at[p, b],
                        gsems[p][b],
                    ).wait()
                    pltpu.sync_copy(
                        gbuf.at[p, b], acc_sh.at[col_v.at[g * G + b]],
                        add=True,
                    )
            return carry

        lax.fori_loop(0, NG // 2, body, 0)
        plsc.subcore_barrier()
        pltpu.sync_copy(acc_sh.at[pl.ds(s * RPT, RPT)], stage_v)

        @pl.when(c == 0)
        def _():
            pltpu.sync_copy(stage_v, out0.at[pl.ds(s * RPT, RPT)])

        @pl.when(c == 1)
        def _():
            pltpu.sync_copy(stage_v, out1.at[pl.ds(s * RPT, RPT)])

    return _mp


def _sc_msgpass32(h, ei):
    return _make_msgpass(H1, False)(h, ei)


def _sc_msgpass16(h, ei):
    return _make_msgpass(H2, True)(h, ei)


def _tc1_body(x_ref, w1_ref, d0_ref, d1_ref, h1s_ref, dinv_ref):
    deg = d0_ref[...] + d1_ref[...] + 1.0
    dinv = lax.rsqrt(deg)
    h = jnp.dot(x_ref[...], w1_ref[...], preferred_element_type=jnp.float32)
    h1s_ref[...] = h * dinv
    dinv_ref[...] = dinv


def _tc1(x, W1, d0, d1):
    return pl.pallas_call(
        _tc1_body,
        out_shape=[
            jax.ShapeDtypeStruct((N, H1), jnp.float32),
            jax.ShapeDtypeStruct((N, 1), jnp.float32),
        ],
    )(x, W1, d0, d1)


def _tc2_body(a0_ref, a1_ref, h1s_ref, dinv_ref, b1_ref, w2_ref, h2s_ref):
    dinv = dinv_ref[...]
    h1 = jnp.maximum(
        (a0_ref[...] + a1_ref[...] + h1s_ref[...]) * dinv + b1_ref[...], 0.0
    )
    h2s_ref[...] = (
        jnp.dot(h1, w2_ref[...], preferred_element_type=jnp.float32) * dinv
    )


def _tc2(a0, a1, h1s, dinv, b1, W2):
    return pl.pallas_call(
        _tc2_body,
        out_shape=jax.ShapeDtypeStruct((N, H2), jnp.float32),
    )(a0, a1, h1s, dinv, b1, W2)


def _tc3_body(a0_ref, a1_ref, h2s_ref, dinv_ref, b2_ref, wihT_ref, bg_ref,
              wout_ref, bout_ref, out_ref):
    h2 = jnp.maximum(
        (a0_ref[...] + a1_ref[...] + h2s_ref[...]) * dinv_ref[...] + b2_ref[...],
        0.0,
    )
    gates = (
        jnp.dot(h2, wihT_ref[...], preferred_element_type=jnp.float32)
        + bg_ref[...]
    )
    i_g = gates[:, 0:HL]
    g_g = gates[:, 2 * HL : 3 * HL]
    o_g = gates[:, 3 * HL : 4 * HL]
    cell = jax.nn.sigmoid(i_g) * jnp.tanh(g_g)
    hh = jax.nn.sigmoid(o_g) * jnp.tanh(cell)
    out_ref[...] = (
        jnp.dot(hh, wout_ref[...], preferred_element_type=jnp.float32)
        + bout_ref[...]
    )


def _tc3(a0, a1, h2s, dinv, b2, wihT, bg, W_out, b_out):
    return pl.pallas_call(
        _tc3_body,
        out_shape=jax.ShapeDtypeStruct((N, 1), jnp.float32),
    )(a0, a1, h2s, dinv, b2, wihT, bg, W_out, b_out)


def kernel(x, edge_index, W1, b1, W2, b2, W_ih, W_hh, b_ih, b_hh, W_out, b_out):
    ei = edge_index.astype(jnp.int32)
    pad_src = jnp.arange(EP - E, dtype=jnp.int32) % N
    pad_dst = (jnp.arange(EP - E, dtype=jnp.int32) % (NP - N)) + N
    ei_pad = jnp.concatenate(
        [ei, jnp.stack([pad_src, pad_dst])], axis=1
    ).reshape(2, NW, CH, K)

    d0, d1 = _sc_degree(ei_pad)
    h1s, dinv = _tc1(x, W1, d0[:N].reshape(N, 1), d1[:N].reshape(N, 1))
    a0, a1 = _sc_msgpass32(h1s, ei_pad)
    h2s = _tc2(a0[:N], a1[:N], h1s, dinv, b1.reshape(1, H1), W2)
    c0, c1 = _sc_msgpass16(h2s, ei_pad)
    out = _tc3(
        c0[:N], c1[:N], h2s, dinv,
        b2.reshape(1, H2), W_ih.T, (b_ih + b_hh).reshape(1, 4 * HL),
        W_out, b_out.reshape(1, 1),
    )
    return out[:, 0]


# speedup vs baseline: 1.5589x; 1.5589x over previous
import functools

import jax
import jax.numpy as jnp
from jax import lax
from jax.experimental import pallas as pl
from jax.experimental.pallas import tpu as pltpu
from jax.experimental.pallas import tpu_sc as plsc

N = 10000
E = 320000
D = 128
H1 = 32
H2 = 16
HL = 8

NC = 2
NS = 16
NW = NC * NS
NP = 10240
RPT = NP // NS
K = 128
CH = 80
EPT = CH * K
EP = NW * EPT
G = 8
NG = CH // G
SR = N // NS

_SC_PARAMS = pltpu.CompilerParams(use_tc_tiling_on_sc=False)


def _sc_mesh():
    return plsc.VectorSubcoreMesh(
        core_axis_name="c", subcore_axis_name="s", num_cores=NC, num_subcores=NS
    )


def _zero_flat(ref, nwords):
    z = jnp.zeros((16,), jnp.float32)

    def body(i, carry):
        ref[pl.ds(i * 16, 16)] = z
        return carry

    lax.fori_loop(0, nwords // 16, body, 0)


def _zero_2d(ref, rows, cols):
    z = jnp.zeros((16,), jnp.float32)

    def body(r, carry):
        for k in range(cols // 16):
            ref[r, pl.ds(k * 16, 16)] = z
        return carry

    lax.fori_loop(0, rows, body, 0)


@functools.cache
def _build_sc_degree():
    return functools.partial(
        pl.kernel,
        out_type=(
            jax.ShapeDtypeStruct((NP,), jnp.float32),
            jax.ShapeDtypeStruct((NP,), jnp.float32),
        ),
        mesh=_sc_mesh(),
        scratch_types=[
            pltpu.VMEM((CH, K), jnp.int32),
            pltpu.VMEM((K,), jnp.float32),
            pltpu.VMEM((RPT,), jnp.float32),
            pltpu.VMEM_SHARED((NP,), jnp.float32),
        ],
        compiler_params=_SC_PARAMS,
    )(_sc_degree_body)


def _sc_degree_body(ei_hbm, out0, out1, col_v, ones_v, stage_v, acc_sh):
    c = lax.axis_index("c")
    s = lax.axis_index("s")
    w = c * NS + s
    pltpu.sync_copy(ei_hbm.at[1, w], col_v)
    one = jnp.ones((16,), jnp.float32)
    for k in range(K // 16):
        ones_v[pl.ds(k * 16, 16)] = one
    _zero_flat(stage_v, RPT)
    pltpu.sync_copy(stage_v, acc_sh.at[pl.ds(s * RPT, RPT)])
    plsc.subcore_barrier()

    def body(j, carry):
        pltpu.sync_copy(ones_v, acc_sh.at[col_v.at[j]], add=True)
        return carry

    lax.fori_loop(0, CH, body, 0)
    plsc.subcore_barrier()
    pltpu.sync_copy(acc_sh.at[pl.ds(s * RPT, RPT)], stage_v)

    @pl.when(c == 0)
    def _():
        pltpu.sync_copy(stage_v, out0.at[pl.ds(s * RPT, RPT)])

    @pl.when(c == 1)
    def _():
        pltpu.sync_copy(stage_v, out1.at[pl.ds(s * RPT, RPT)])


def _sc_degree(ei):
    return _build_sc_degree()(ei)


@functools.cache
def _make_msgpass(F, stage_tbl):
    scratch = [
        pltpu.VMEM((CH, K), jnp.int32),
        pltpu.VMEM((CH, K), jnp.int32),
        pltpu.VMEM((2, G, K, F), jnp.float32),
        pltpu.VMEM((RPT, F), jnp.float32),
        pltpu.VMEM_SHARED((NP, F), jnp.float32),
        [[pltpu.SemaphoreType.DMA] * G, [pltpu.SemaphoreType.DMA] * G],
    ]
    if stage_tbl:
        scratch.append(pltpu.VMEM_SHARED((N, F), jnp.float32))

    @functools.partial(
        pl.kernel,
        out_type=(
            jax.ShapeDtypeStruct((NP, F), jnp.float32),
            jax.ShapeDtypeStruct((NP, F), jnp.float32),
        ),
        mesh=_sc_mesh(),
        scratch_types=scratch,
        compiler_params=_SC_PARAMS,
        name=f"sc_msgpass_f{F}",
    )
    def _mp(h_hbm, ei_hbm, out0, out1, row_v, col_v, gbuf, stage_v, acc_sh,
            gsems, *maybe_tbl):
        c = lax.axis_index("c")
        s = lax.axis_index("s")
        w = c * NS + s
        pltpu.sync_copy(ei_hbm.at[0, w], row_v)
        pltpu.sync_copy(ei_hbm.at[1, w], col_v)
        if stage_tbl:
            tbl = maybe_tbl[0]
            pltpu.sync_copy(
                h_hbm.at[pl.ds(s * SR, SR)], stage_v.at[pl.ds(0, SR)]
            )
            pltpu.sync_copy(
                stage_v.at[pl.ds(0, SR)], tbl.at[pl.ds(s * SR, SR)]
            )
        else:
            tbl = h_hbm
        _zero_2d(stage_v, RPT, F)
        pltpu.sync_copy(stage_v, acc_sh.at[pl.ds(s * RPT, RPT)])
        plsc.subcore_barrier()
        for b in range(G):
            pltpu.async_copy(tbl.at[row_v.at[b]], gbuf.at[0, b], gsems[0][b])

        def body(i, carry):
            for p in (0, 1):
                g = i * 2 + p
                q = 1 - p

                @pl.when(g + 1 < NG)
                def _():
                    for b in range(G):
                        pltpu.async_copy(
                            tbl.at[row_v.at[(g + 1) * G + b]],
                            gbuf.at[q, b], gsems[q][b],
                        )

                for b in range(G):
                    pltpu.make_async_copy(
                        tbl.at[row_v.at[g * G + b]], gbuf.at[p, b],
                        gsems[p][b],
                    ).wait()
                    pltpu.sync_copy(
                        gbuf.at[p, b], acc_sh.at[col_v.at[g * G + b]],
                        add=True,
                    )
            return carry

        lax.fori_loop(0, NG // 2, body, 0)
        plsc.subcore_barrier()
        pltpu.sync_copy(acc_sh.at[pl.ds(s * RPT, RPT)], stage_v)

        @pl.when(c == 0)
        def _():
            pltpu.sync_copy(stage_v, out0.at[pl.ds(s * RPT, RPT)])

        @pl.when(c == 1)
        def _():
            pltpu.sync_copy(stage_v, out1.at[pl.ds(s * RPT, RPT)])

    return _mp


def _sc_msgpass32(h, ei):
    return _make_msgpass(H1, False)(h, ei)


def _sc_msgpass16(h, ei):
    return _make_msgpass(H2, False)(h, ei)


def _tc1_body(x4_ref, w1bd_ref, dinv4_ref, h1sp_ref):
    h = jnp.dot(x4_ref[...], w1bd_ref[...], preferred_element_type=jnp.float32)
    h1sp_ref[...] = h * dinv4_ref[...]


def _tc1(x4, W1bd, dinv4):
    return pl.pallas_call(
        _tc1_body,
        out_shape=jax.ShapeDtypeStruct((N // 4, 4 * H1), jnp.float32),
    )(x4, W1bd, dinv4)


def _tc2_body(a0_ref, a1_ref, h1sp_ref, dinv4_ref, b1p_ref, w2bd_ref,
              dinv4h_ref, h2sp_ref):
    agg = a0_ref[...][0 : N // 4] + a1_ref[...][0 : N // 4]
    h1 = jnp.maximum(
        (agg + h1sp_ref[...]) * dinv4_ref[...] + b1p_ref[...], 0.0
    )
    h2sp_ref[...] = (
        jnp.dot(h1, w2bd_ref[...], preferred_element_type=jnp.float32)
        * dinv4h_ref[...]
    )


def _tc2(a0p, a1p, h1sp, dinv4, b1p, W2bd, dinv4h):
    return pl.pallas_call(
        _tc2_body,
        out_shape=jax.ShapeDtypeStruct((N // 4, 4 * H2), jnp.float32),
    )(a0p, a1p, h1sp, dinv4, b1p, W2bd, dinv4h)


def _tc3_body(c0_ref, c1_ref, h2sp_ref, dinv8_ref, b2p_ref, wihbd_ref,
              bgp_ref, wout_ref, bout_ref, out_ref):
    agg = c0_ref[...][0 : N // 8] + c1_ref[...][0 : N // 8]
    h2 = jnp.maximum(
        (agg + h2sp_ref[...]) * dinv8_ref[...] + b2p_ref[...], 0.0
    )
    gates = (
        jnp.dot(h2, wihbd_ref[...], preferred_element_type=jnp.float32)
        + bgp_ref[...]
    )
    gp = 8 * HL
    sig_i = jax.nn.sigmoid(gates[:, 0:gp])
    tah_g = jnp.tanh(gates[:, 2 * gp : 3 * gp])
    sig_o = jax.nn.sigmoid(gates[:, 3 * gp : 4 * gp])
    hh = sig_o * jnp.tanh(sig_i * tah_g)
    out_ref[...] = (
        jnp.dot(hh, wout_ref[...], preferred_element_type=jnp.float32)
        + bout_ref[...]
    )


def _tc3(c0p, c1p, h2sp8, dinv8, b2p, Wihbd, bgp, W_out, b_out):
    return pl.pallas_call(
        _tc3_body,
        out_shape=jax.ShapeDtypeStruct((N // 8, HL), jnp.float32),
    )(c0p, c1p, h2sp8, dinv8, b2p, Wihbd, bgp, W_out, b_out)


def kernel(x, edge_index, W1, b1, W2, b2, W_ih, W_hh, b_ih, b_hh, W_out, b_out):
    ei = edge_index.astype(jnp.int32)
    pad_src = jnp.arange(EP - E, dtype=jnp.int32) % N
    pad_dst = (jnp.arange(EP - E, dtype=jnp.int32) % (NP - N)) + N
    ei_pad = jnp.concatenate(
        [ei, jnp.stack([pad_src, pad_dst])], axis=1
    ).reshape(2, NW, CH, K)

    eye4 = jnp.eye(4, dtype=jnp.float32)
    eye8 = jnp.eye(8, dtype=jnp.float32)
    W1bd = jnp.kron(eye4, W1)
    W2bd = jnp.kron(eye4, W2)
    Wihbd = jnp.concatenate(
        [jnp.kron(eye8, W_ih.T[:, 8 * t : 8 * t + 8]) for t in range(4)],
        axis=1,
    )
    bg = b_ih + b_hh
    bgp = jnp.concatenate(
        [jnp.tile(bg[8 * t : 8 * t + 8], 8) for t in range(4)]
    ).reshape(1, 32 * HL)
    Woutbd = jnp.kron(eye8, W_out)

    d0, d1 = _sc_degree(ei_pad)
    dinv1d = lax.rsqrt(d0[:N] + d1[:N] + 1.0)
    dinv4 = jnp.broadcast_to(dinv1d[:, None], (N, H1)).reshape(N // 4, 4 * H1)
    dinv4h = jnp.broadcast_to(dinv1d[:, None], (N, H2)).reshape(N // 4, 4 * H2)
    dinv8 = jnp.broadcast_to(dinv1d[:, None], (N, H2)).reshape(N // 8, 8 * H2)

    h1sp = _tc1(x.reshape(N // 4, 4 * D), W1bd, dinv4)
    a0, a1 = _sc_msgpass32(h1sp.reshape(N, H1), ei_pad)
    h2sp = _tc2(
        a0.reshape(NP // 4, 4 * H1), a1.reshape(NP // 4, 4 * H1),
        h1sp, dinv4, jnp.tile(b1, 4).reshape(1, 4 * H1), W2bd, dinv4h,
    )
    c0, c1 = _sc_msgpass16(h2sp.reshape(N, H2), ei_pad)
    out = _tc3(
        c0.reshape(NP // 8, 8 * H2), c1.reshape(NP // 8, 8 * H2),
        h2sp.reshape(N // 8, 8 * H2), dinv8,
        jnp.tile(b2, 8).reshape(1, 8 * H2), Wihbd, bgp,
        Woutbd, b_out.reshape(1, 1),
    )
    return out.reshape(N)


# speedup vs baseline: 1.6131x; 1.0348x over previous
import functools

import jax
import jax.numpy as jnp
from jax import lax
from jax.experimental import pallas as pl
from jax.experimental.pallas import tpu as pltpu
from jax.experimental.pallas import tpu_sc as plsc

N = 10000
E = 320000
D = 128
H1 = 32
H2 = 16
HL = 8

NC = 2
NS = 16
NW = NC * NS
NP = 10240
RPT = NP // NS
K = 128
CH = 80
EPT = CH * K
EP = NW * EPT
G = 8
NG = CH // G
SR = N // NS

_SC_PARAMS = pltpu.CompilerParams(use_tc_tiling_on_sc=False)


def _sc_mesh():
    return plsc.VectorSubcoreMesh(
        core_axis_name="c", subcore_axis_name="s", num_cores=NC, num_subcores=NS
    )


def _zero_flat(ref, nwords):
    z = jnp.zeros((16,), jnp.float32)

    def body(i, carry):
        ref[pl.ds(i * 16, 16)] = z
        return carry

    lax.fori_loop(0, nwords // 16, body, 0)


def _zero_2d(ref, rows, cols):
    z = jnp.zeros((16,), jnp.float32)

    def body(r, carry):
        for k in range(cols // 16):
            ref[r, pl.ds(k * 16, 16)] = z
        return carry

    lax.fori_loop(0, rows, body, 0)


@functools.cache
def _build_sc_degree():
    return functools.partial(
        pl.kernel,
        out_type=(
            jax.ShapeDtypeStruct((NP,), jnp.float32),
            jax.ShapeDtypeStruct((NP,), jnp.float32),
        ),
        mesh=_sc_mesh(),
        scratch_types=[
            pltpu.VMEM((CH, K), jnp.int32),
            pltpu.VMEM((K,), jnp.float32),
            pltpu.VMEM((RPT,), jnp.float32),
            pltpu.VMEM_SHARED((NP,), jnp.float32),
        ],
        compiler_params=_SC_PARAMS,
    )(_sc_degree_body)


def _sc_degree_body(ei_hbm, out0, out1, col_v, ones_v, stage_v, acc_sh):
    c = lax.axis_index("c")
    s = lax.axis_index("s")
    w = c * NS + s
    pltpu.sync_copy(ei_hbm.at[1, w], col_v)
    one = jnp.ones((16,), jnp.float32)
    for k in range(K // 16):
        ones_v[pl.ds(k * 16, 16)] = one
    _zero_flat(stage_v, RPT)
    pltpu.sync_copy(stage_v, acc_sh.at[pl.ds(s * RPT, RPT)])
    plsc.subcore_barrier()

    def body(j, carry):
        pltpu.sync_copy(ones_v, acc_sh.at[col_v.at[j]], add=True)
        return carry

    lax.fori_loop(0, CH, body, 0)
    plsc.subcore_barrier()
    pltpu.sync_copy(acc_sh.at[pl.ds(s * RPT, RPT)], stage_v)

    @pl.when(c == 0)
    def _():
        pltpu.sync_copy(stage_v, out0.at[pl.ds(s * RPT, RPT)])

    @pl.when(c == 1)
    def _():
        pltpu.sync_copy(stage_v, out1.at[pl.ds(s * RPT, RPT)])


def _sc_degree(ei):
    return _build_sc_degree()(ei)


@functools.cache
def _make_msgpass(F, stage_tbl):
    scratch = [
        pltpu.VMEM((CH, K), jnp.int32),
        pltpu.VMEM((CH, K), jnp.int32),
        pltpu.VMEM((2, G, K, F), jnp.float32),
        pltpu.VMEM((RPT, F), jnp.float32),
        pltpu.VMEM_SHARED((NP, F), jnp.float32),
        [[pltpu.SemaphoreType.DMA] * G, [pltpu.SemaphoreType.DMA] * G],
    ]
    if stage_tbl:
        scratch.append(pltpu.VMEM_SHARED((N, F), jnp.float32))

    @functools.partial(
        pl.kernel,
        out_type=(
            jax.ShapeDtypeStruct((NP, F), jnp.float32),
            jax.ShapeDtypeStruct((NP, F), jnp.float32),
        ),
        mesh=_sc_mesh(),
        scratch_types=scratch,
        compiler_params=_SC_PARAMS,
        name=f"sc_msgpass_f{F}",
    )
    def _mp(h_hbm, ei_hbm, out0, out1, row_v, col_v, gbuf, stage_v, acc_sh,
            gsems, *maybe_tbl):
        c = lax.axis_index("c")
        s = lax.axis_index("s")
        w = c * NS + s
        pltpu.sync_copy(ei_hbm.at[0, w], row_v)
        pltpu.sync_copy(ei_hbm.at[1, w], col_v)
        if stage_tbl:
            tbl = maybe_tbl[0]
            pltpu.sync_copy(
                h_hbm.at[pl.ds(s * SR, SR)], stage_v.at[pl.ds(0, SR)]
            )
            pltpu.sync_copy(
                stage_v.at[pl.ds(0, SR)], tbl.at[pl.ds(s * SR, SR)]
            )
        else:
            tbl = h_hbm
        _zero_2d(stage_v, RPT, F)
        pltpu.sync_copy(stage_v, acc_sh.at[pl.ds(s * RPT, RPT)])
        plsc.subcore_barrier()
        for b in range(G):
            pltpu.async_copy(tbl.at[row_v.at[b]], gbuf.at[0, b], gsems[0][b])

        def body(i, carry):
            for p in (0, 1):
                g = i * 2 + p
                q = 1 - p

                @pl.when(g + 1 < NG)
                def _():
                    for b in range(G):
                        pltpu.async_copy(
                            tbl.at[row_v.at[(g + 1) * G + b]],
                            gbuf.at[q, b], gsems[q][b],
                        )

                for b in range(G):
                    pltpu.make_async_copy(
                        tbl.at[row_v.at[g * G + b]], gbuf.at[p, b],
                        gsems[p][b],
                    ).wait()
                    pltpu.sync_copy(
                        gbuf.at[p, b], acc_sh.at[col_v.at[g * G + b]],
                        add=True,
                    )
            return carry

        lax.fori_loop(0, NG // 2, body, 0)
        plsc.subcore_barrier()
        pltpu.sync_copy(acc_sh.at[pl.ds(s * RPT, RPT)], stage_v)

        @pl.when(c == 0)
        def _():
            pltpu.sync_copy(stage_v, out0.at[pl.ds(s * RPT, RPT)])

        @pl.when(c == 1)
        def _():
            pltpu.sync_copy(stage_v, out1.at[pl.ds(s * RPT, RPT)])

    return _mp


def _sc_msgpass32(h, ei):
    return _make_msgpass(H1, False)(h, ei)


def _sc_msgpass16(h, ei):
    return _make_msgpass(H2, False)(h, ei)


def _tc1_body(x4_ref, w1bd_ref, h1sp_ref):
    h1sp_ref[...] = jnp.dot(
        x4_ref[...], w1bd_ref[...], preferred_element_type=jnp.float32
    )


def _tc1(x4, W1bd):
    return pl.pallas_call(
        _tc1_body,
        out_shape=jax.ShapeDtypeStruct((N // 4, 4 * H1), jnp.float32),
    )(x4, W1bd)


def _tc2_body(a0_ref, a1_ref, h1sp_ref, dinv4_ref, b1p_ref, w2bd_ref,
              dinv4h_ref, h2sp_ref):
    agg = a0_ref[...][0 : N // 4] + a1_ref[...][0 : N // 4]
    h1 = jnp.maximum(
        (agg + h1sp_ref[...]) * dinv4_ref[...] + b1p_ref[...], 0.0
    )
    h2sp_ref[...] = (
        jnp.dot(h1, w2bd_ref[...], preferred_element_type=jnp.float32)
        * dinv4h_ref[...]
    )


def _tc2(a0p, a1p, h1sp, dinv4, b1p, W2bd, dinv4h):
    return pl.pallas_call(
        _tc2_body,
        out_shape=jax.ShapeDtypeStruct((N // 4, 4 * H2), jnp.float32),
    )(a0p, a1p, h1sp, dinv4, b1p, W2bd, dinv4h)


def _tc3_body(c0_ref, c1_ref, h2sp_ref, dinv8_ref, b2p_ref, wihbd_ref,
              bgp_ref, wout_ref, bout_ref, out_ref):
    agg = c0_ref[...][0 : N // 8] + c1_ref[...][0 : N // 8]
    h2 = jnp.maximum(
        (agg + h2sp_ref[...]) * dinv8_ref[...] + b2p_ref[...], 0.0
    )
    gates = (
        jnp.dot(h2, wihbd_ref[...], preferred_element_type=jnp.float32)
        + bgp_ref[...]
    )
    gp = 8 * HL
    sig_i = jax.nn.sigmoid(gates[:, 0:gp])
    tah_g = jnp.tanh(gates[:, 2 * gp : 3 * gp])
    sig_o = jax.nn.sigmoid(gates[:, 3 * gp : 4 * gp])
    hh = sig_o * jnp.tanh(sig_i * tah_g)
    out_ref[...] = (
        jnp.dot(hh, wout_ref[...], preferred_element_type=jnp.float32)
        + bout_ref[...]
    )


def _tc3(c0p, c1p, h2sp8, dinv8, b2p, Wihbd, bgp, W_out, b_out):
    return pl.pallas_call(
        _tc3_body,
        out_shape=jax.ShapeDtypeStruct((N // 8, HL), jnp.float32),
    )(c0p, c1p, h2sp8, dinv8, b2p, Wihbd, bgp, W_out, b_out)


def kernel(x, edge_index, W1, b1, W2, b2, W_ih, W_hh, b_ih, b_hh, W_out, b_out):
    ei = edge_index.astype(jnp.int32)
    pad_src = jnp.arange(EP - E, dtype=jnp.int32) % N
    pad_dst = (jnp.arange(EP - E, dtype=jnp.int32) % (NP - N)) + N
    ei_pad = jnp.concatenate(
        [ei, jnp.stack([pad_src, pad_dst])], axis=1
    ).reshape(2, NW, CH, K)

    eye4 = jnp.eye(4, dtype=jnp.float32)
    eye8 = jnp.eye(8, dtype=jnp.float32)
    W1bd = jnp.kron(eye4, W1)
    W2bd = jnp.kron(eye4, W2)
    Wihbd = jnp.concatenate(
        [jnp.kron(eye8, W_ih.T[:, 8 * t : 8 * t + 8]) for t in range(4)],
        axis=1,
    )
    bg = b_ih + b_hh
    bgp = jnp.concatenate(
        [jnp.tile(bg[8 * t : 8 * t + 8], 8) for t in range(4)]
    ).reshape(1, 32 * HL)
    Woutbd = jnp.kron(eye8, W_out)

    d0, d1 = _sc_degree(ei_pad)
    h1raw = _tc1(x.reshape(N // 4, 4 * D), W1bd)
    dinv1d = lax.rsqrt(d0[:N] + d1[:N] + 1.0)
    dinv4 = jnp.broadcast_to(dinv1d[:, None], (N, H1)).reshape(N // 4, 4 * H1)
    dinv4h = jnp.broadcast_to(dinv1d[:, None], (N, H2)).reshape(N // 4, 4 * H2)
    dinv8 = jnp.broadcast_to(dinv1d[:, None], (N, H2)).reshape(N // 8, 8 * H2)

    h1sp = h1raw * dinv4
    a0, a1 = _sc_msgpass32(h1sp.reshape(N, H1), ei_pad)
    h2sp = _tc2(
        a0.reshape(NP // 4, 4 * H1), a1.reshape(NP // 4, 4 * H1),
        h1sp, dinv4, jnp.tile(b1, 4).reshape(1, 4 * H1), W2bd, dinv4h,
    )
    c0, c1 = _sc_msgpass16(h2sp.reshape(N, H2), ei_pad)
    out = _tc3(
        c0.reshape(NP // 8, 8 * H2), c1.reshape(NP // 8, 8 * H2),
        h2sp.reshape(N // 8, 8 * H2), dinv8,
        jnp.tile(b2, 8).reshape(1, 8 * H2), Wihbd, bgp,
        Woutbd, b_out.reshape(1, 1),
    )
    return out.reshape(N)
